# Initial kernel scaffold; baseline (speedup 1.0000x reference)
import jax
import jax.numpy as jnp
from jax.experimental import pallas as pl


def kernel(x, edge_index, W, b):
    raise NotImplementedError("write your pallas kernel here")


# speedup vs baseline: 35.2906x; 35.2906x over previous
import functools

import jax
import jax.numpy as jnp
from jax import lax
from jax.experimental import pallas as pl
from jax.experimental.pallas import tpu as pltpu
from jax.experimental.pallas import tpu_sc as plsc

NC = 2
NS = 16
NW = NC * NS
CHUNK = 128
LANES = 16


def _round_up(v, m):
    return (v + m - 1) // m * m


def _sc_degree(dst3, acc_rows):
    nchunk = dst3.shape[1]
    rpt = acc_rows // NS
    mesh = plsc.VectorSubcoreMesh(core_axis_name="c", subcore_axis_name="s")

    @functools.partial(
        pl.kernel,
        out_type=jax.ShapeDtypeStruct((NC, acc_rows, LANES), jnp.float32),
        mesh=mesh,
        scratch_types=[
            pltpu.VMEM((nchunk, CHUNK), jnp.int32),
            pltpu.VMEM((CHUNK, LANES), jnp.float32),
            pltpu.VMEM((CHUNK, LANES), jnp.float32),
            pltpu.VMEM_SHARED((acc_rows, LANES), jnp.float32),
            pltpu.SemaphoreType.DMA,
        ],
        compiler_params=pltpu.CompilerParams(use_tc_tiling_on_sc=False),
    )
    def deg_kernel(dst_hbm, deg_out, dstbuf, ones_v, zeros_v, deg_s, sem):
        c = lax.axis_index("c")
        s = lax.axis_index("s")
        wid = c * NS + s

        def fill(i, _):
            ones_v[i, :] = jnp.ones((LANES,), jnp.float32)
            zeros_v[i, :] = jnp.zeros((LANES,), jnp.float32)
            return 0

        lax.fori_loop(0, CHUNK, fill, 0)

        def zero_slab(r, _):
            pltpu.sync_copy(
                zeros_v, deg_s.at[pl.ds(s * rpt + r * CHUNK, CHUNK)]
            )
            return 0

        lax.fori_loop(0, rpt // CHUNK, zero_slab, 0)
        plsc.subcore_barrier()

        pltpu.sync_copy(dst_hbm.at[wid], dstbuf)

        def step(j, _):
            pltpu.sync_copy(ones_v, deg_s.at[dstbuf.at[j]], add=True)
            return 0

        lax.fori_loop(0, nchunk, step, 0)
        plsc.subcore_barrier()

        pltpu.sync_copy(
            deg_s.at[pl.ds(s * rpt, rpt)],
            deg_out.at[c, pl.ds(s * rpt, rpt)],
        )

    return deg_kernel(dst3)


def _sc_messages(y, src3, dst3, acc_rows, out_ch):
    nchunk = src3.shape[1]
    rpt = acc_rows // NS
    mesh = plsc.VectorSubcoreMesh(core_axis_name="c", subcore_axis_name="s")

    @functools.partial(
        pl.kernel,
        out_type=jax.ShapeDtypeStruct((NC, acc_rows, out_ch), jnp.float32),
        mesh=mesh,
        scratch_types=[
            pltpu.VMEM((nchunk, CHUNK), jnp.int32),
            pltpu.VMEM((nchunk, CHUNK), jnp.int32),
            pltpu.VMEM((CHUNK, out_ch), jnp.float32),
            pltpu.VMEM((CHUNK, out_ch), jnp.float32),
            pltpu.VMEM_SHARED((acc_rows, out_ch), jnp.float32),
            pltpu.SemaphoreType.DMA,
        ],
        compiler_params=pltpu.CompilerParams(use_tc_tiling_on_sc=False),
    )
    def msg_kernel(y_hbm, src_hbm, dst_hbm, acc_out,
                   srcbuf, dstbuf, rows_v, zeros_v, acc_s, sem):
        c = lax.axis_index("c")
        s = lax.axis_index("s")
        wid = c * NS + s
        lanes_per_row = out_ch // LANES

        def fill(t, _):
            zeros_v[t // lanes_per_row,
                    pl.ds((t % lanes_per_row) * LANES, LANES)] = (
                jnp.zeros((LANES,), jnp.float32))
            return 0

        lax.fori_loop(0, CHUNK * lanes_per_row, fill, 0)

        def zero_slab(r, _):
            pltpu.sync_copy(
                zeros_v, acc_s.at[pl.ds(s * rpt + r * CHUNK, CHUNK)]
            )
            return 0

        lax.fori_loop(0, rpt // CHUNK, zero_slab, 0)
        plsc.subcore_barrier()

        pltpu.sync_copy(src_hbm.at[wid], srcbuf)
        pltpu.sync_copy(dst_hbm.at[wid], dstbuf)

        def step(j, _):
            pltpu.async_copy(y_hbm.at[srcbuf.at[j]], rows_v, sem).wait()
            pltpu.sync_copy(rows_v, acc_s.at[dstbuf.at[j]], add=True)
            return 0

        lax.fori_loop(0, nchunk, step, 0)
        plsc.subcore_barrier()

        pltpu.sync_copy(
            acc_s.at[pl.ds(s * rpt, rpt)],
            acc_out.at[c, pl.ds(s * rpt, rpt)],
        )

    return msg_kernel(y, src3, dst3)


def _tc_prep(x, w, deg_part, n):
    out_ch = w.shape[1]

    def body(x_ref, w_ref, deg_ref, y_ref, dinv_ref):
        deg = deg_ref[0, :n, 0:1] + deg_ref[1, :n, 0:1]
        dinv = 1.0 / jnp.sqrt(deg)
        xw = jnp.dot(x_ref[...], w_ref[...],
                     preferred_element_type=jnp.float32)
        y_ref[...] = xw * dinv
        dinv_ref[...] = dinv

    return pl.pallas_call(
        body,
        out_shape=[
            jax.ShapeDtypeStruct((n, out_ch), jnp.float32),
            jax.ShapeDtypeStruct((n, 1), jnp.float32),
        ],
    )(x, w, deg_part)


def _tc_final(acc_part, dinv, b2, n):
    out_ch = acc_part.shape[2]

    def body(acc_ref, dinv_ref, b_ref, o_ref):
        p = acc_ref[0, :n, :] + acc_ref[1, :n, :]
        o_ref[...] = p * dinv_ref[...] + b_ref[...]

    return pl.pallas_call(
        body,
        out_shape=jax.ShapeDtypeStruct((n, out_ch), jnp.float32),
    )(acc_part, dinv, b2)


def kernel(x, edge_index, W, b):
    n = x.shape[0]
    out_ch = W.shape[1]
    e = edge_index.shape[1]

    loop = jnp.arange(n, dtype=edge_index.dtype)
    src = jnp.concatenate([edge_index[0], loop])
    dst = jnp.concatenate([edge_index[1], loop])

    acc_rows = _round_up(n + 1, NS * CHUNK)
    e2 = _round_up(e + n, NW * CHUNK)
    pad = e2 - (e + n)
    pad_ar = jnp.arange(pad, dtype=edge_index.dtype)
    src = jnp.concatenate([src, pad_ar % n])
    dst = jnp.concatenate([dst, n + pad_ar % (acc_rows - n)])
    nchunk = e2 // (NW * CHUNK)
    src3 = src.reshape(NW, nchunk, CHUNK)
    dst3 = dst.reshape(NW, nchunk, CHUNK)

    deg_part = _sc_degree(dst3, acc_rows)
    y, dinv = _tc_prep(x, W, deg_part, n)
    acc_part = _sc_messages(y, src3, dst3, acc_rows, out_ch)
    out = _tc_final(acc_part, dinv, b.reshape(1, out_ch), n)
    return out


# speedup vs baseline: 40.4504x; 1.1462x over previous
import functools

import jax
import jax.numpy as jnp
from jax import lax
from jax.experimental import pallas as pl
from jax.experimental.pallas import tpu as pltpu
from jax.experimental.pallas import tpu_sc as plsc

NC = 2
NS = 16
NW = NC * NS
CHUNK = 128
LANES = 16


def _round_up(v, m):
    return (v + m - 1) // m * m


def _sc_degree(dst3, acc_rows):
    nchunk = dst3.shape[1]
    rpt = acc_rows // NS
    mesh = plsc.VectorSubcoreMesh(core_axis_name="c", subcore_axis_name="s")

    @functools.partial(
        pl.kernel,
        out_type=jax.ShapeDtypeStruct((NC, acc_rows, LANES), jnp.float32),
        mesh=mesh,
        scratch_types=[
            pltpu.VMEM((nchunk, CHUNK), jnp.int32),
            pltpu.VMEM((CHUNK, LANES), jnp.float32),
            pltpu.VMEM((CHUNK, LANES), jnp.float32),
            pltpu.VMEM_SHARED((acc_rows, LANES), jnp.float32),
            pltpu.SemaphoreType.DMA,
        ],
        compiler_params=pltpu.CompilerParams(use_tc_tiling_on_sc=False),
    )
    def deg_kernel(dst_hbm, deg_out, dstbuf, ones_v, zeros_v, deg_s, sem):
        c = lax.axis_index("c")
        s = lax.axis_index("s")
        wid = c * NS + s

        def fill(i, _):
            ones_v[i, :] = jnp.ones((LANES,), jnp.float32)
            zeros_v[i, :] = jnp.zeros((LANES,), jnp.float32)
            return 0

        lax.fori_loop(0, CHUNK, fill, 0)

        def zero_slab(r, _):
            pltpu.sync_copy(
                zeros_v, deg_s.at[pl.ds(s * rpt + r * CHUNK, CHUNK)]
            )
            return 0

        lax.fori_loop(0, rpt // CHUNK, zero_slab, 0)
        plsc.subcore_barrier()

        pltpu.sync_copy(dst_hbm.at[wid], dstbuf)

        group = 7
        assert nchunk % group == 0

        def grp(g, _):
            def fire(j, _):
                pltpu.async_copy(ones_v, deg_s.at[dstbuf.at[j]], sem,
                                 add=True)
                return 0

            lax.fori_loop(g * group, (g + 1) * group, fire, 0)

            def drain(j, _):
                pltpu.make_async_copy(
                    ones_v, deg_s.at[dstbuf.at[j]], sem).wait()
                return 0

            lax.fori_loop(g * group, (g + 1) * group, drain, 0)
            return 0

        lax.fori_loop(0, nchunk // group, grp, 0)
        plsc.subcore_barrier()

        pltpu.sync_copy(
            deg_s.at[pl.ds(s * rpt, rpt)],
            deg_out.at[c, pl.ds(s * rpt, rpt)],
        )

    return deg_kernel(dst3)


def _sc_messages(y, src3, dst3, acc_rows, out_ch):
    nchunk = src3.shape[1]
    rpt = acc_rows // NS
    mesh = plsc.VectorSubcoreMesh(core_axis_name="c", subcore_axis_name="s")

    @functools.partial(
        pl.kernel,
        out_type=jax.ShapeDtypeStruct((NC, acc_rows, out_ch), jnp.float32),
        mesh=mesh,
        scratch_types=[
            pltpu.VMEM((nchunk, CHUNK), jnp.int32),
            pltpu.VMEM((nchunk, CHUNK), jnp.int32),
            pltpu.VMEM((CHUNK, out_ch), jnp.float32),
            pltpu.VMEM((CHUNK, out_ch), jnp.float32),
            pltpu.VMEM((CHUNK, out_ch), jnp.float32),
            pltpu.VMEM_SHARED((acc_rows, out_ch), jnp.float32),
            pltpu.SemaphoreType.DMA,
            pltpu.SemaphoreType.DMA,
        ],
        compiler_params=pltpu.CompilerParams(use_tc_tiling_on_sc=False),
    )
    def msg_kernel(y_hbm, src_hbm, dst_hbm, acc_out,
                   srcbuf, dstbuf, rows_a, rows_b, zeros_v, acc_s,
                   sem_a, sem_b):
        c = lax.axis_index("c")
        s = lax.axis_index("s")
        wid = c * NS + s
        lanes_per_row = out_ch // LANES

        def fill(t, _):
            zeros_v[t // lanes_per_row,
                    pl.ds((t % lanes_per_row) * LANES, LANES)] = (
                jnp.zeros((LANES,), jnp.float32))
            return 0

        lax.fori_loop(0, CHUNK * lanes_per_row, fill, 0)

        def zero_slab(r, _):
            pltpu.sync_copy(
                zeros_v, acc_s.at[pl.ds(s * rpt + r * CHUNK, CHUNK)]
            )
            return 0

        lax.fori_loop(0, rpt // CHUNK, zero_slab, 0)
        plsc.subcore_barrier()

        pltpu.sync_copy(src_hbm.at[wid], srcbuf)
        pltpu.sync_copy(dst_hbm.at[wid], dstbuf)

        def gather(j, buf, sem):
            pltpu.async_copy(y_hbm.at[srcbuf.at[j]], buf, sem)

        def gwait(j, buf, sem):
            pltpu.make_async_copy(y_hbm.at[srcbuf.at[j]], buf, sem).wait()

        def scatter(j, buf):
            pltpu.sync_copy(buf, acc_s.at[dstbuf.at[j]], add=True)

        gather(0, rows_a, sem_a)

        def pair(g, _):
            j0 = 2 * g
            gwait(j0, rows_a, sem_a)
            gather(j0 + 1, rows_b, sem_b)
            scatter(j0, rows_a)
            gwait(j0 + 1, rows_b, sem_b)
            gather(j0 + 2, rows_a, sem_a)
            scatter(j0 + 1, rows_b)
            return 0

        lax.fori_loop(0, nchunk // 2 - 1, pair, 0)
        j0 = nchunk - 2
        gwait(j0, rows_a, sem_a)
        gather(j0 + 1, rows_b, sem_b)
        scatter(j0, rows_a)
        gwait(j0 + 1, rows_b, sem_b)
        scatter(j0 + 1, rows_b)
        plsc.subcore_barrier()

        pltpu.sync_copy(
            acc_s.at[pl.ds(s * rpt, rpt)],
            acc_out.at[c, pl.ds(s * rpt, rpt)],
        )

    return msg_kernel(y, src3, dst3)


def _tc_prep(x, w, deg_part, n):
    out_ch = w.shape[1]

    def body(x_ref, w_ref, deg_ref, y_ref, dinv_ref):
        deg = deg_ref[0, :n, 0:1] + deg_ref[1, :n, 0:1]
        dinv = 1.0 / jnp.sqrt(deg)
        xw = jnp.dot(x_ref[...], w_ref[...],
                     preferred_element_type=jnp.float32)
        y_ref[...] = xw * dinv
        dinv_ref[...] = dinv

    return pl.pallas_call(
        body,
        out_shape=[
            jax.ShapeDtypeStruct((n, out_ch), jnp.float32),
            jax.ShapeDtypeStruct((n, 1), jnp.float32),
        ],
    )(x, w, deg_part)


def _tc_final(acc_part, dinv, b2, n):
    out_ch = acc_part.shape[2]

    def body(acc_ref, dinv_ref, b_ref, o_ref):
        p = acc_ref[0, :n, :] + acc_ref[1, :n, :]
        o_ref[...] = p * dinv_ref[...] + b_ref[...]

    return pl.pallas_call(
        body,
        out_shape=jax.ShapeDtypeStruct((n, out_ch), jnp.float32),
    )(acc_part, dinv, b2)


def kernel(x, edge_index, W, b):
    n = x.shape[0]
    out_ch = W.shape[1]
    e = edge_index.shape[1]

    loop = jnp.arange(n, dtype=edge_index.dtype)
    src = jnp.concatenate([edge_index[0], loop])
    dst = jnp.concatenate([edge_index[1], loop])

    acc_rows = _round_up(n + 1, NS * CHUNK)
    e2 = _round_up(_round_up(e + n, NW * CHUNK) // (NW * CHUNK), 14) * (
        NW * CHUNK)
    pad = e2 - (e + n)
    pad_ar = jnp.arange(pad, dtype=edge_index.dtype)
    src = jnp.concatenate([src, pad_ar % n])
    dst = jnp.concatenate([dst, n + pad_ar % (acc_rows - n)])
    nchunk = e2 // (NW * CHUNK)
    src3 = src.reshape(NW, nchunk, CHUNK)
    dst3 = dst.reshape(NW, nchunk, CHUNK)

    deg_part = _sc_degree(dst3, acc_rows)
    y, dinv = _tc_prep(x, W, deg_part, n)
    acc_part = _sc_messages(y, src3, dst3, acc_rows, out_ch)
    out = _tc_final(acc_part, dinv, b.reshape(1, out_ch), n)
    return out


# speedup vs baseline: 48.4281x; 1.1972x over previous
import functools

import jax
import jax.numpy as jnp
from jax import lax
from jax.experimental import pallas as pl
from jax.experimental.pallas import tpu as pltpu
from jax.experimental.pallas import tpu_sc as plsc

NC = 2
NS = 16
NW = NC * NS
CHUNK = 128
LANES = 16


def _round_up(v, m):
    return (v + m - 1) // m * m


def _sc_degree(dst3, acc_rows):
    nchunk = dst3.shape[1]
    rpt = acc_rows // NS
    mesh = plsc.VectorSubcoreMesh(core_axis_name="c", subcore_axis_name="s")

    @functools.partial(
        pl.kernel,
        out_type=jax.ShapeDtypeStruct((NC, acc_rows, LANES), jnp.float32),
        mesh=mesh,
        scratch_types=[
            pltpu.VMEM((nchunk, CHUNK), jnp.int32),
            pltpu.VMEM((CHUNK, LANES), jnp.float32),
            pltpu.VMEM((CHUNK, LANES), jnp.float32),
            pltpu.VMEM_SHARED((acc_rows, LANES), jnp.float32),
            pltpu.SemaphoreType.DMA,
        ],
        compiler_params=pltpu.CompilerParams(use_tc_tiling_on_sc=False),
    )
    def deg_kernel(dst_hbm, deg_out, dstbuf, ones_v, zeros_v, deg_s, sem):
        c = lax.axis_index("c")
        s = lax.axis_index("s")
        wid = c * NS + s

        def fill(i, _):
            ones_v[i, :] = jnp.ones((LANES,), jnp.float32)
            zeros_v[i, :] = jnp.zeros((LANES,), jnp.float32)
            return 0

        lax.fori_loop(0, CHUNK, fill, 0)

        def zero_slab(r, _):
            pltpu.sync_copy(
                zeros_v, deg_s.at[pl.ds(s * rpt + r * CHUNK, CHUNK)]
            )
            return 0

        lax.fori_loop(0, rpt // CHUNK, zero_slab, 0)
        plsc.subcore_barrier()

        pltpu.sync_copy(dst_hbm.at[wid], dstbuf)

        group = 7
        assert nchunk % group == 0

        def grp(g, _):
            def fire(j, _):
                pltpu.async_copy(ones_v, deg_s.at[dstbuf.at[j]], sem,
                                 add=True)
                return 0

            lax.fori_loop(g * group, (g + 1) * group, fire, 0)

            def drain(j, _):
                pltpu.make_async_copy(
                    ones_v, deg_s.at[dstbuf.at[j]], sem).wait()
                return 0

            lax.fori_loop(g * group, (g + 1) * group, drain, 0)
            return 0

        lax.fori_loop(0, nchunk // group, grp, 0)
        plsc.subcore_barrier()

        pltpu.sync_copy(
            deg_s.at[pl.ds(s * rpt, rpt)],
            deg_out.at[c, pl.ds(s * rpt, rpt)],
        )

    return deg_kernel(dst3)


def _sc_messages(y, src3, dst3, acc_rows, out_ch):
    nchunk = src3.shape[1]
    rpt = acc_rows // NS
    mesh = plsc.VectorSubcoreMesh(core_axis_name="c", subcore_axis_name="s")

    @functools.partial(
        pl.kernel,
        out_type=jax.ShapeDtypeStruct((NC, acc_rows, out_ch), jnp.float32),
        mesh=mesh,
        scratch_types=[
            pltpu.VMEM((nchunk, CHUNK), jnp.int32),
            pltpu.VMEM((nchunk, CHUNK), jnp.int32),
            pltpu.VMEM((3 * CHUNK, out_ch), jnp.float32),
            pltpu.VMEM((3 * CHUNK, out_ch), jnp.float32),
            pltpu.VMEM((CHUNK, out_ch), jnp.float32),
            pltpu.VMEM_SHARED((acc_rows, out_ch), jnp.float32),
            pltpu.SemaphoreType.DMA,
            pltpu.SemaphoreType.DMA,
            pltpu.SemaphoreType.DMA,
            pltpu.SemaphoreType.DMA,
        ],
        compiler_params=pltpu.CompilerParams(use_tc_tiling_on_sc=False),
    )
    def msg_kernel(y_hbm, src_hbm, dst_hbm, acc_out,
                   srcbuf, dstbuf, rows_a, rows_b, zeros_v, acc_s,
                   sem_ga, sem_gb, sem_sa, sem_sb):
        c = lax.axis_index("c")
        s = lax.axis_index("s")
        wid = c * NS + s
        lanes_per_row = out_ch // LANES

        def fill(t, _):
            zeros_v[t // lanes_per_row,
                    pl.ds((t % lanes_per_row) * LANES, LANES)] = (
                jnp.zeros((LANES,), jnp.float32))
            return 0

        lax.fori_loop(0, CHUNK * lanes_per_row, fill, 0)

        def zero_slab(r, _):
            pltpu.sync_copy(
                zeros_v, acc_s.at[pl.ds(s * rpt + r * CHUNK, CHUNK)]
            )
            return 0

        lax.fori_loop(0, rpt // CHUNK, zero_slab, 0)
        plsc.subcore_barrier()

        pltpu.sync_copy(src_hbm.at[wid], srcbuf)
        pltpu.sync_copy(dst_hbm.at[wid], dstbuf)

        K = 3
        nsuper = nchunk // K

        def fire_gathers(js, buf, sem):
            for i in range(K):
                pltpu.async_copy(
                    y_hbm.at[srcbuf.at[js * K + i]],
                    buf.at[pl.ds(i * CHUNK, CHUNK)], sem)

        def drain_gathers(js, buf, sem):
            for i in range(K):
                pltpu.make_async_copy(
                    y_hbm.at[srcbuf.at[js * K + i]],
                    buf.at[pl.ds(i * CHUNK, CHUNK)], sem).wait()

        def fire_scatters(js, buf, sem):
            for i in range(K):
                pltpu.async_copy(
                    buf.at[pl.ds(i * CHUNK, CHUNK)],
                    acc_s.at[dstbuf.at[js * K + i]], sem, add=True)

        def drain_scatters(js, buf, sem):
            for i in range(K):
                pltpu.make_async_copy(
                    buf.at[pl.ds(i * CHUNK, CHUNK)],
                    acc_s.at[dstbuf.at[js * K + i]], sem).wait()

        fire_gathers(0, rows_a, sem_ga)

        def pair(g, _):
            js0 = 2 * g
            drain_gathers(js0, rows_a, sem_ga)
            fire_gathers(js0 + 1, rows_b, sem_gb)
            fire_scatters(js0, rows_a, sem_sa)
            drain_scatters(js0, rows_a, sem_sa)
            fire_gathers(js0 + 2, rows_a, sem_ga)
            drain_gathers(js0 + 1, rows_b, sem_gb)
            fire_scatters(js0 + 1, rows_b, sem_sb)
            drain_scatters(js0 + 1, rows_b, sem_sb)
            return 0

        lax.fori_loop(0, nsuper // 2 - 1, pair, 0)
        js0 = nsuper - 2
        drain_gathers(js0, rows_a, sem_ga)
        fire_gathers(js0 + 1, rows_b, sem_gb)
        fire_scatters(js0, rows_a, sem_sa)
        drain_scatters(js0, rows_a, sem_sa)
        drain_gathers(js0 + 1, rows_b, sem_gb)
        fire_scatters(js0 + 1, rows_b, sem_sb)
        drain_scatters(js0 + 1, rows_b, sem_sb)
        plsc.subcore_barrier()

        pltpu.sync_copy(
            acc_s.at[pl.ds(s * rpt, rpt)],
            acc_out.at[c, pl.ds(s * rpt, rpt)],
        )

    return msg_kernel(y, src3, dst3)


def _tc_prep(x, w, deg_part, n):
    out_ch = w.shape[1]

    def body(x_ref, w_ref, deg_ref, y_ref, dinv_ref):
        deg = deg_ref[0, :n, 0:1] + deg_ref[1, :n, 0:1]
        dinv = 1.0 / jnp.sqrt(deg)
        xw = jnp.dot(x_ref[...], w_ref[...],
                     preferred_element_type=jnp.float32)
        y_ref[...] = xw * dinv
        dinv_ref[...] = dinv

    return pl.pallas_call(
        body,
        out_shape=[
            jax.ShapeDtypeStruct((n, out_ch), jnp.float32),
            jax.ShapeDtypeStruct((n, 1), jnp.float32),
        ],
    )(x, w, deg_part)


def _tc_final(acc_part, dinv, b2, n):
    out_ch = acc_part.shape[2]

    def body(acc_ref, dinv_ref, b_ref, o_ref):
        p = acc_ref[0, :n, :] + acc_ref[1, :n, :]
        o_ref[...] = p * dinv_ref[...] + b_ref[...]

    return pl.pallas_call(
        body,
        out_shape=jax.ShapeDtypeStruct((n, out_ch), jnp.float32),
    )(acc_part, dinv, b2)


def kernel(x, edge_index, W, b):
    n = x.shape[0]
    out_ch = W.shape[1]
    e = edge_index.shape[1]

    loop = jnp.arange(n, dtype=edge_index.dtype)
    src = jnp.concatenate([edge_index[0], loop])
    dst = jnp.concatenate([edge_index[1], loop])

    acc_rows = _round_up(n + 1, NS * CHUNK)
    e2 = _round_up(_round_up(e + n, NW * CHUNK) // (NW * CHUNK), 42) * (
        NW * CHUNK)
    pad = e2 - (e + n)
    pad_ar = jnp.arange(pad, dtype=edge_index.dtype)
    src = jnp.concatenate([src, pad_ar % n])
    dst = jnp.concatenate([dst, n + pad_ar % (acc_rows - n)])
    nchunk = e2 // (NW * CHUNK)
    src3 = src.reshape(NW, nchunk, CHUNK)
    dst3 = dst.reshape(NW, nchunk, CHUNK)

    deg_part = _sc_degree(dst3, acc_rows)
    y, dinv = _tc_prep(x, W, deg_part, n)
    acc_part = _sc_messages(y, src3, dst3, acc_rows, out_ch)
    out = _tc_final(acc_part, dinv, b.reshape(1, out_ch), n)
    return out


# speedup vs baseline: 50.9579x; 1.0522x over previous
import functools

import jax
import jax.numpy as jnp
from jax import lax
from jax.experimental import pallas as pl
from jax.experimental.pallas import tpu as pltpu
from jax.experimental.pallas import tpu_sc as plsc

NC = 2
NS = 16
NW = NC * NS
CHUNK = 128
LANES = 16


def _round_up(v, m):
    return (v + m - 1) // m * m


def _sc_degree(dst2, acc_rows):
    nch = dst2.shape[0]
    base = nch // NW
    extra = nch % NW
    rpt = acc_rows // NS
    group = 6
    assert base % group == 0
    mesh = plsc.VectorSubcoreMesh(core_axis_name="c", subcore_axis_name="s")

    @functools.partial(
        pl.kernel,
        out_type=jax.ShapeDtypeStruct((NC, acc_rows, LANES), jnp.float32),
        mesh=mesh,
        scratch_types=[
            pltpu.VMEM((base + 1, CHUNK), jnp.int32),
            pltpu.VMEM((CHUNK, LANES), jnp.float32),
            pltpu.VMEM((CHUNK, LANES), jnp.float32),
            pltpu.VMEM_SHARED((acc_rows, LANES), jnp.float32),
            pltpu.SemaphoreType.DMA,
        ],
        compiler_params=pltpu.CompilerParams(use_tc_tiling_on_sc=False),
    )
    def deg_kernel(dst_hbm, deg_out, dstbuf, ones_v, zeros_v, deg_s, sem):
        c = lax.axis_index("c")
        s = lax.axis_index("s")
        wid = c * NS + s

        def fill(i, _):
            ones_v[i, :] = jnp.ones((LANES,), jnp.float32)
            zeros_v[i, :] = jnp.zeros((LANES,), jnp.float32)
            return 0

        lax.fori_loop(0, CHUNK, fill, 0)

        def zero_slab(r, _):
            pltpu.sync_copy(
                zeros_v, deg_s.at[pl.ds(s * rpt + r * CHUNK, CHUNK)]
            )
            return 0

        lax.fori_loop(0, rpt // CHUNK, zero_slab, 0)
        plsc.subcore_barrier()

        pltpu.sync_copy(dst_hbm.at[pl.ds(wid * base, base)],
                        dstbuf.at[pl.ds(0, base)])

        @pl.when(wid < extra)
        def _():
            pltpu.sync_copy(dst_hbm.at[pl.ds(NW * base + wid, 1)],
                            dstbuf.at[pl.ds(base, 1)])

        def grp(g, _):
            def fire(j, _):
                pltpu.async_copy(ones_v, deg_s.at[dstbuf.at[j]], sem,
                                 add=True)
                return 0

            lax.fori_loop(g * group, (g + 1) * group, fire, 0)

            def drain(j, _):
                pltpu.make_async_copy(
                    ones_v, deg_s.at[dstbuf.at[j]], sem).wait()
                return 0

            lax.fori_loop(g * group, (g + 1) * group, drain, 0)
            return 0

        lax.fori_loop(0, base // group, grp, 0)

        @pl.when(wid < extra)
        def _():
            pltpu.sync_copy(ones_v, deg_s.at[dstbuf.at[base]], add=True)

        plsc.subcore_barrier()

        pltpu.sync_copy(
            deg_s.at[pl.ds(s * rpt, rpt)],
            deg_out.at[c, pl.ds(s * rpt, rpt)],
        )

    return deg_kernel(dst2)


def _sc_messages(y, src2, dst2, acc_rows, out_ch):
    nch = src2.shape[0]
    base = nch // NW
    extra = nch % NW
    rpt = acc_rows // NS
    K = 3
    nsuper = base // K
    assert base % K == 0 and nsuper % 2 == 0
    mesh = plsc.VectorSubcoreMesh(core_axis_name="c", subcore_axis_name="s")

    @functools.partial(
        pl.kernel,
        out_type=jax.ShapeDtypeStruct((NC, acc_rows, out_ch), jnp.float32),
        mesh=mesh,
        scratch_types=[
            pltpu.VMEM((base + 1, CHUNK), jnp.int32),
            pltpu.VMEM((base + 1, CHUNK), jnp.int32),
            pltpu.VMEM((K * CHUNK, out_ch), jnp.float32),
            pltpu.VMEM((K * CHUNK, out_ch), jnp.float32),
            pltpu.VMEM((CHUNK, out_ch), jnp.float32),
            pltpu.VMEM_SHARED((acc_rows, out_ch), jnp.float32),
            pltpu.SemaphoreType.DMA,
            pltpu.SemaphoreType.DMA,
            pltpu.SemaphoreType.DMA,
            pltpu.SemaphoreType.DMA,
        ],
        compiler_params=pltpu.CompilerParams(use_tc_tiling_on_sc=False),
    )
    def msg_kernel(y_hbm, src_hbm, dst_hbm, acc_out,
                   srcbuf, dstbuf, rows_a, rows_b, zeros_v, acc_s,
                   sem_ga, sem_gb, sem_sa, sem_sb):
        c = lax.axis_index("c")
        s = lax.axis_index("s")
        wid = c * NS + s
        lanes_per_row = out_ch // LANES

        def fill(t, _):
            zeros_v[t // lanes_per_row,
                    pl.ds((t % lanes_per_row) * LANES, LANES)] = (
                jnp.zeros((LANES,), jnp.float32))
            return 0

        lax.fori_loop(0, CHUNK * lanes_per_row, fill, 0)

        def zero_slab(r, _):
            pltpu.sync_copy(
                zeros_v, acc_s.at[pl.ds(s * rpt + r * CHUNK, CHUNK)]
            )
            return 0

        lax.fori_loop(0, rpt // CHUNK, zero_slab, 0)
        plsc.subcore_barrier()

        pltpu.sync_copy(src_hbm.at[pl.ds(wid * base, base)],
                        srcbuf.at[pl.ds(0, base)])
        pltpu.sync_copy(dst_hbm.at[pl.ds(wid * base, base)],
                        dstbuf.at[pl.ds(0, base)])

        @pl.when(wid < extra)
        def _():
            pltpu.sync_copy(src_hbm.at[pl.ds(NW * base + wid, 1)],
                            srcbuf.at[pl.ds(base, 1)])
            pltpu.sync_copy(dst_hbm.at[pl.ds(NW * base + wid, 1)],
                            dstbuf.at[pl.ds(base, 1)])

        def fire_gathers(js, buf, sem):
            for i in range(K):
                pltpu.async_copy(
                    y_hbm.at[srcbuf.at[js * K + i]],
                    buf.at[pl.ds(i * CHUNK, CHUNK)], sem)

        def drain_gathers(js, buf, sem):
            for i in range(K):
                pltpu.make_async_copy(
                    y_hbm.at[srcbuf.at[js * K + i]],
                    buf.at[pl.ds(i * CHUNK, CHUNK)], sem).wait()

        def fire_scatters(js, buf, sem):
            for i in range(K):
                pltpu.async_copy(
                    buf.at[pl.ds(i * CHUNK, CHUNK)],
                    acc_s.at[dstbuf.at[js * K + i]], sem, add=True)

        def drain_scatters(js, buf, sem):
            for i in range(K):
                pltpu.make_async_copy(
                    buf.at[pl.ds(i * CHUNK, CHUNK)],
                    acc_s.at[dstbuf.at[js * K + i]], sem).wait()

        fire_gathers(0, rows_a, sem_ga)

        def pair(g, _):
            js0 = 2 * g
            drain_gathers(js0, rows_a, sem_ga)
            fire_gathers(js0 + 1, rows_b, sem_gb)
            fire_scatters(js0, rows_a, sem_sa)
            drain_scatters(js0, rows_a, sem_sa)
            fire_gathers(js0 + 2, rows_a, sem_ga)
            drain_gathers(js0 + 1, rows_b, sem_gb)
            fire_scatters(js0 + 1, rows_b, sem_sb)
            drain_scatters(js0 + 1, rows_b, sem_sb)
            return 0

        lax.fori_loop(0, nsuper // 2 - 1, pair, 0)
        js0 = nsuper - 2
        drain_gathers(js0, rows_a, sem_ga)
        fire_gathers(js0 + 1, rows_b, sem_gb)
        fire_scatters(js0, rows_a, sem_sa)
        drain_scatters(js0, rows_a, sem_sa)
        drain_gathers(js0 + 1, rows_b, sem_gb)
        fire_scatters(js0 + 1, rows_b, sem_sb)
        drain_scatters(js0 + 1, rows_b, sem_sb)

        @pl.when(wid < extra)
        def _():
            pltpu.async_copy(
                y_hbm.at[srcbuf.at[base]],
                rows_a.at[pl.ds(0, CHUNK)], sem_ga).wait()
            pltpu.sync_copy(rows_a.at[pl.ds(0, CHUNK)],
                            acc_s.at[dstbuf.at[base]], add=True)

        plsc.subcore_barrier()

        pltpu.sync_copy(
            acc_s.at[pl.ds(s * rpt, rpt)],
            acc_out.at[c, pl.ds(s * rpt, rpt)],
        )

    return msg_kernel(y, src2, dst2)


def _tc_prep(x, w, deg_part, n):
    out_ch = w.shape[1]

    def body(x_ref, w_ref, deg_ref, y_ref, dinv_ref):
        deg = deg_ref[0, :n, 0:1] + deg_ref[1, :n, 0:1]
        dinv = 1.0 / jnp.sqrt(deg + 1.0)
        xw = jnp.dot(x_ref[...], w_ref[...],
                     preferred_element_type=jnp.float32)
        y_ref[...] = xw * dinv
        dinv_ref[...] = dinv

    return pl.pallas_call(
        body,
        out_shape=[
            jax.ShapeDtypeStruct((n, out_ch), jnp.float32),
            jax.ShapeDtypeStruct((n, 1), jnp.float32),
        ],
    )(x, w, deg_part)


def _tc_final(acc_part, y, dinv, b2, n):
    out_ch = acc_part.shape[2]

    def body(acc_ref, y_ref, dinv_ref, b_ref, o_ref):
        p = acc_ref[0, :n, :] + acc_ref[1, :n, :] + y_ref[...]
        o_ref[...] = p * dinv_ref[...] + b_ref[...]

    return pl.pallas_call(
        body,
        out_shape=jax.ShapeDtypeStruct((n, out_ch), jnp.float32),
    )(acc_part, y, dinv, b2)


def kernel(x, edge_index, W, b):
    n = x.shape[0]
    out_ch = W.shape[1]
    e = edge_index.shape[1]
    assert e % CHUNK == 0

    acc_rows = _round_up(n, NS * CHUNK)
    nch = e // CHUNK
    src2 = edge_index[0].reshape(nch, CHUNK)
    dst2 = edge_index[1].reshape(nch, CHUNK)

    deg_part = _sc_degree(dst2, acc_rows)
    y, dinv = _tc_prep(x, W, deg_part, n)
    acc_part = _sc_messages(y, src2, dst2, acc_rows, out_ch)
    out = _tc_final(acc_part, y, dinv, b.reshape(1, out_ch), n)
    return out


# speedup vs baseline: 54.8958x; 1.0773x over previous
import functools

import jax
import jax.numpy as jnp
from jax import lax
from jax.experimental import pallas as pl
from jax.experimental.pallas import tpu as pltpu
from jax.experimental.pallas import tpu_sc as plsc

NC = 2
NS = 16
NW = NC * NS
CHUNK = 128
LANES = 16


def _round_up(v, m):
    return (v + m - 1) // m * m


def _sc_degree(edge3, acc_rows):
    nch = edge3.shape[1]
    base = nch // NW
    extra = nch % NW
    rpt = acc_rows // NS
    group = 6
    assert base % group == 0
    mesh = plsc.VectorSubcoreMesh(core_axis_name="c", subcore_axis_name="s")

    @functools.partial(
        pl.kernel,
        out_type=jax.ShapeDtypeStruct((NC, acc_rows, LANES), jnp.float32),
        mesh=mesh,
        scratch_types=[
            pltpu.VMEM((base + 1, CHUNK), jnp.int32),
            pltpu.VMEM((CHUNK, LANES), jnp.float32),
            pltpu.VMEM((CHUNK, LANES), jnp.float32),
            pltpu.VMEM_SHARED((acc_rows, LANES), jnp.float32),
            pltpu.SemaphoreType.DMA,
        ],
        compiler_params=pltpu.CompilerParams(use_tc_tiling_on_sc=False),
    )
    def deg_kernel(edge_hbm, deg_out, dstbuf, ones_v, zeros_v, deg_s, sem):
        c = lax.axis_index("c")
        s = lax.axis_index("s")
        wid = c * NS + s

        def fill(i, _):
            ones_v[i, :] = jnp.ones((LANES,), jnp.float32)
            zeros_v[i, :] = jnp.zeros((LANES,), jnp.float32)
            return 0

        lax.fori_loop(0, CHUNK, fill, 0)

        def zero_slab(r, _):
            pltpu.sync_copy(
                zeros_v, deg_s.at[pl.ds(s * rpt + r * CHUNK, CHUNK)]
            )
            return 0

        lax.fori_loop(0, rpt // CHUNK, zero_slab, 0)
        plsc.subcore_barrier()

        pltpu.sync_copy(edge_hbm.at[1, pl.ds(wid * base, base)],
                        dstbuf.at[pl.ds(0, base)])

        @pl.when(wid < extra)
        def _():
            pltpu.sync_copy(edge_hbm.at[1, pl.ds(NW * base + wid, 1)],
                            dstbuf.at[pl.ds(base, 1)])

        def grp(g, _):
            def fire(j, _):
                pltpu.async_copy(ones_v, deg_s.at[dstbuf.at[j]], sem,
                                 add=True)
                return 0

            lax.fori_loop(g * group, (g + 1) * group, fire, 0)

            def drain(j, _):
                pltpu.make_async_copy(
                    ones_v, deg_s.at[dstbuf.at[j]], sem).wait()
                return 0

            lax.fori_loop(g * group, (g + 1) * group, drain, 0)
            return 0

        lax.fori_loop(0, base // group, grp, 0)

        @pl.when(wid < extra)
        def _():
            pltpu.sync_copy(ones_v, deg_s.at[dstbuf.at[base]], add=True)

        plsc.subcore_barrier()

        pltpu.sync_copy(
            deg_s.at[pl.ds(s * rpt, rpt)],
            deg_out.at[c, pl.ds(s * rpt, rpt)],
        )

    return deg_kernel(edge3)


def _sc_messages(y, edge3, acc_rows, out_ch):
    nch = edge3.shape[1]
    base = nch // NW
    extra = nch % NW
    rpt = acc_rows // NS
    K = 3
    nsuper = base // K
    assert base % K == 0 and nsuper % 2 == 0
    mesh = plsc.VectorSubcoreMesh(core_axis_name="c", subcore_axis_name="s")

    @functools.partial(
        pl.kernel,
        out_type=jax.ShapeDtypeStruct((NC, acc_rows, out_ch), jnp.float32),
        mesh=mesh,
        scratch_types=[
            pltpu.VMEM((base + 1, CHUNK), jnp.int32),
            pltpu.VMEM((base + 1, CHUNK), jnp.int32),
            pltpu.VMEM((K * CHUNK, out_ch), jnp.float32),
            pltpu.VMEM((K * CHUNK, out_ch), jnp.float32),
            pltpu.VMEM((CHUNK, out_ch), jnp.float32),
            pltpu.VMEM_SHARED((acc_rows, out_ch), jnp.float32),
            pltpu.SemaphoreType.DMA,
            pltpu.SemaphoreType.DMA,
            pltpu.SemaphoreType.DMA,
            pltpu.SemaphoreType.DMA,
        ],
        compiler_params=pltpu.CompilerParams(use_tc_tiling_on_sc=False),
    )
    def msg_kernel(y_hbm, edge_hbm, acc_out,
                   srcbuf, dstbuf, rows_a, rows_b, zeros_v, acc_s,
                   sem_ga, sem_gb, sem_sa, sem_sb):
        c = lax.axis_index("c")
        s = lax.axis_index("s")
        wid = c * NS + s
        lanes_per_row = out_ch // LANES

        def fill(t, _):
            zeros_v[t // lanes_per_row,
                    pl.ds((t % lanes_per_row) * LANES, LANES)] = (
                jnp.zeros((LANES,), jnp.float32))
            return 0

        lax.fori_loop(0, CHUNK * lanes_per_row, fill, 0)

        def zero_slab(r, _):
            pltpu.sync_copy(
                zeros_v, acc_s.at[pl.ds(s * rpt + r * CHUNK, CHUNK)]
            )
            return 0

        lax.fori_loop(0, rpt // CHUNK, zero_slab, 0)
        plsc.subcore_barrier()

        pltpu.sync_copy(edge_hbm.at[0, pl.ds(wid * base, base)],
                        srcbuf.at[pl.ds(0, base)])
        pltpu.sync_copy(edge_hbm.at[1, pl.ds(wid * base, base)],
                        dstbuf.at[pl.ds(0, base)])

        @pl.when(wid < extra)
        def _():
            pltpu.sync_copy(edge_hbm.at[0, pl.ds(NW * base + wid, 1)],
                            srcbuf.at[pl.ds(base, 1)])
            pltpu.sync_copy(edge_hbm.at[1, pl.ds(NW * base + wid, 1)],
                            dstbuf.at[pl.ds(base, 1)])

        def fire_gathers(js, buf, sem):
            for i in range(K):
                pltpu.async_copy(
                    y_hbm.at[srcbuf.at[js * K + i]],
                    buf.at[pl.ds(i * CHUNK, CHUNK)], sem)

        def drain_gathers(js, buf, sem):
            for i in range(K):
                pltpu.make_async_copy(
                    y_hbm.at[srcbuf.at[js * K + i]],
                    buf.at[pl.ds(i * CHUNK, CHUNK)], sem).wait()

        def fire_scatters(js, buf, sem):
            for i in range(K):
                pltpu.async_copy(
                    buf.at[pl.ds(i * CHUNK, CHUNK)],
                    acc_s.at[dstbuf.at[js * K + i]], sem, add=True)

        def drain_scatters(js, buf, sem):
            for i in range(K):
                pltpu.make_async_copy(
                    buf.at[pl.ds(i * CHUNK, CHUNK)],
                    acc_s.at[dstbuf.at[js * K + i]], sem).wait()

        fire_gathers(0, rows_a, sem_ga)

        def pair(g, _):
            js0 = 2 * g
            drain_gathers(js0, rows_a, sem_ga)
            fire_gathers(js0 + 1, rows_b, sem_gb)
            fire_scatters(js0, rows_a, sem_sa)
            drain_scatters(js0, rows_a, sem_sa)
            fire_gathers(js0 + 2, rows_a, sem_ga)
            drain_gathers(js0 + 1, rows_b, sem_gb)
            fire_scatters(js0 + 1, rows_b, sem_sb)
            drain_scatters(js0 + 1, rows_b, sem_sb)
            return 0

        lax.fori_loop(0, nsuper // 2 - 1, pair, 0)
        js0 = nsuper - 2
        drain_gathers(js0, rows_a, sem_ga)
        fire_gathers(js0 + 1, rows_b, sem_gb)
        fire_scatters(js0, rows_a, sem_sa)
        drain_scatters(js0, rows_a, sem_sa)
        drain_gathers(js0 + 1, rows_b, sem_gb)
        fire_scatters(js0 + 1, rows_b, sem_sb)
        drain_scatters(js0 + 1, rows_b, sem_sb)

        @pl.when(wid < extra)
        def _():
            pltpu.async_copy(
                y_hbm.at[srcbuf.at[base]],
                rows_a.at[pl.ds(0, CHUNK)], sem_ga).wait()
            pltpu.sync_copy(rows_a.at[pl.ds(0, CHUNK)],
                            acc_s.at[dstbuf.at[base]], add=True)

        plsc.subcore_barrier()

        pltpu.sync_copy(
            acc_s.at[pl.ds(s * rpt, rpt)],
            acc_out.at[c, pl.ds(s * rpt, rpt)],
        )

    return msg_kernel(y, edge3)


def _tc_matmul(x, w):
    n = x.shape[0]
    out_ch = w.shape[1]

    def body(x_ref, w_ref, xw_ref):
        xw_ref[...] = jnp.dot(x_ref[...], w_ref[...],
                              preferred_element_type=jnp.float32)

    return pl.pallas_call(
        body,
        out_shape=jax.ShapeDtypeStruct((n, out_ch), jnp.float32),
    )(x, w)


def _tc_scale(xw, deg_part, n):
    out_ch = xw.shape[1]

    def body(xw_ref, deg_ref, y_ref, dinv_ref):
        deg = deg_ref[0, :n, 0:1] + deg_ref[1, :n, 0:1]
        dinv = 1.0 / jnp.sqrt(deg + 1.0)
        y_ref[...] = xw_ref[...] * dinv
        dinv_ref[...] = dinv

    return pl.pallas_call(
        body,
        out_shape=[
            jax.ShapeDtypeStruct((n, out_ch), jnp.float32),
            jax.ShapeDtypeStruct((n, 1), jnp.float32),
        ],
    )(xw, deg_part)


def _tc_final(acc_part, y, dinv, b2, n):
    out_ch = acc_part.shape[2]

    def body(acc_ref, y_ref, dinv_ref, b_ref, o_ref):
        p = acc_ref[0, :n, :] + acc_ref[1, :n, :] + y_ref[...]
        o_ref[...] = p * dinv_ref[...] + b_ref[...]

    return pl.pallas_call(
        body,
        out_shape=jax.ShapeDtypeStruct((n, out_ch), jnp.float32),
    )(acc_part, y, dinv, b2)


def kernel(x, edge_index, W, b):
    n = x.shape[0]
    out_ch = W.shape[1]
    e = edge_index.shape[1]
    assert e % CHUNK == 0

    acc_rows = _round_up(n, NS * CHUNK)
    nch = e // CHUNK
    edge3 = edge_index.reshape(2, nch, CHUNK)

    deg_part = _sc_degree(edge3, acc_rows)
    xw = _tc_matmul(x, W)
    y, dinv = _tc_scale(xw, deg_part, n)
    acc_part = _sc_messages(y, edge3, acc_rows, out_ch)
    out = _tc_final(acc_part, y, dinv, b.reshape(1, out_ch), n)
    return out


# speedup vs baseline: 55.6885x; 1.0144x over previous
import functools

import jax
import jax.numpy as jnp
from jax import lax
from jax.experimental import pallas as pl
from jax.experimental.pallas import tpu as pltpu
from jax.experimental.pallas import tpu_sc as plsc

NC = 2
NS = 16
NW = NC * NS
CHUNK = 128
LANES = 16


def _round_up(v, m):
    return (v + m - 1) // m * m


def _sc_degree(edge3, acc_rows):
    nch = edge3.shape[1]
    base = nch // NW
    extra = nch % NW
    rpt = acc_rows // NS
    group = 6
    assert base % group == 0
    mesh = plsc.VectorSubcoreMesh(core_axis_name="c", subcore_axis_name="s")

    @functools.partial(
        pl.kernel,
        out_type=jax.ShapeDtypeStruct((NC, acc_rows, LANES), jnp.float32),
        mesh=mesh,
        scratch_types=[
            pltpu.VMEM((base + 1, CHUNK), jnp.int32),
            pltpu.VMEM((CHUNK, LANES), jnp.float32),
            pltpu.VMEM((CHUNK, LANES), jnp.float32),
            pltpu.VMEM_SHARED((acc_rows, LANES), jnp.float32),
            pltpu.SemaphoreType.DMA,
        ],
        compiler_params=pltpu.CompilerParams(use_tc_tiling_on_sc=False),
    )
    def deg_kernel(edge_hbm, deg_out, dstbuf, ones_v, zeros_v, deg_s, sem):
        c = lax.axis_index("c")
        s = lax.axis_index("s")
        wid = c * NS + s

        def fill(i, _):
            ones_v[i, :] = jnp.ones((LANES,), jnp.float32)
            zeros_v[i, :] = jnp.zeros((LANES,), jnp.float32)
            return 0

        lax.fori_loop(0, CHUNK, fill, 0)

        def zero_slab(r, _):
            pltpu.sync_copy(
                zeros_v, deg_s.at[pl.ds(s * rpt + r * CHUNK, CHUNK)]
            )
            return 0

        lax.fori_loop(0, rpt // CHUNK, zero_slab, 0)
        plsc.subcore_barrier()

        pltpu.sync_copy(edge_hbm.at[1, pl.ds(wid * base, base)],
                        dstbuf.at[pl.ds(0, base)])

        @pl.when(wid < extra)
        def _():
            pltpu.sync_copy(edge_hbm.at[1, pl.ds(NW * base + wid, 1)],
                            dstbuf.at[pl.ds(base, 1)])

        def grp(g, _):
            def fire(j, _):
                pltpu.async_copy(ones_v, deg_s.at[dstbuf.at[j]], sem,
                                 add=True)
                return 0

            lax.fori_loop(g * group, (g + 1) * group, fire, 0)

            def drain(j, _):
                pltpu.make_async_copy(
                    ones_v, deg_s.at[dstbuf.at[j]], sem).wait()
                return 0

            lax.fori_loop(g * group, (g + 1) * group, drain, 0)
            return 0

        lax.fori_loop(0, base // group, grp, 0)

        @pl.when(wid < extra)
        def _():
            pltpu.sync_copy(ones_v, deg_s.at[dstbuf.at[base]], add=True)

        plsc.subcore_barrier()

        pltpu.sync_copy(
            deg_s.at[pl.ds(s * rpt, rpt)],
            deg_out.at[c, pl.ds(s * rpt, rpt)],
        )

    return deg_kernel(edge3)


def _sc_messages(y, edge3, acc_rows, out_ch):
    nch = edge3.shape[1]
    base = nch // NW
    extra = nch % NW
    rpt = acc_rows // NS
    K = 2
    nsuper = base // K
    assert base % K == 0 and nsuper % 3 == 0 and nsuper >= 6
    mesh = plsc.VectorSubcoreMesh(core_axis_name="c", subcore_axis_name="s")

    @functools.partial(
        pl.kernel,
        out_type=jax.ShapeDtypeStruct((NC, acc_rows, out_ch), jnp.float32),
        mesh=mesh,
        scratch_types=[
            pltpu.VMEM((base + 1, CHUNK), jnp.int32),
            pltpu.VMEM((base + 1, CHUNK), jnp.int32),
            pltpu.VMEM((K * CHUNK, out_ch), jnp.float32),
            pltpu.VMEM((K * CHUNK, out_ch), jnp.float32),
            pltpu.VMEM((K * CHUNK, out_ch), jnp.float32),
            pltpu.VMEM((CHUNK, out_ch), jnp.float32),
            pltpu.VMEM_SHARED((acc_rows, out_ch), jnp.float32),
            pltpu.SemaphoreType.DMA,
            pltpu.SemaphoreType.DMA,
            pltpu.SemaphoreType.DMA,
            pltpu.SemaphoreType.DMA,
            pltpu.SemaphoreType.DMA,
            pltpu.SemaphoreType.DMA,
        ],
        compiler_params=pltpu.CompilerParams(use_tc_tiling_on_sc=False),
    )
    def msg_kernel(y_hbm, edge_hbm, acc_out,
                   srcbuf, dstbuf, rows_a, rows_b, rows_c, zeros_v, acc_s,
                   sem_ga, sem_gb, sem_gc, sem_sa, sem_sb, sem_sc):
        c = lax.axis_index("c")
        s = lax.axis_index("s")
        wid = c * NS + s
        lanes_per_row = out_ch // LANES

        def fill(t, _):
            zeros_v[t // lanes_per_row,
                    pl.ds((t % lanes_per_row) * LANES, LANES)] = (
                jnp.zeros((LANES,), jnp.float32))
            return 0

        lax.fori_loop(0, CHUNK * lanes_per_row, fill, 0)

        def zero_slab(r, _):
            pltpu.sync_copy(
                zeros_v, acc_s.at[pl.ds(s * rpt + r * CHUNK, CHUNK)]
            )
            return 0

        lax.fori_loop(0, rpt // CHUNK, zero_slab, 0)
        plsc.subcore_barrier()

        pltpu.sync_copy(edge_hbm.at[0, pl.ds(wid * base, base)],
                        srcbuf.at[pl.ds(0, base)])
        pltpu.sync_copy(edge_hbm.at[1, pl.ds(wid * base, base)],
                        dstbuf.at[pl.ds(0, base)])

        @pl.when(wid < extra)
        def _():
            pltpu.sync_copy(edge_hbm.at[0, pl.ds(NW * base + wid, 1)],
                            srcbuf.at[pl.ds(base, 1)])
            pltpu.sync_copy(edge_hbm.at[1, pl.ds(NW * base + wid, 1)],
                            dstbuf.at[pl.ds(base, 1)])

        def fire_gathers(js, buf, sem):
            for i in range(K):
                pltpu.async_copy(
                    y_hbm.at[srcbuf.at[js * K + i]],
                    buf.at[pl.ds(i * CHUNK, CHUNK)], sem)

        def drain_gathers(js, buf, sem):
            for i in range(K):
                pltpu.make_async_copy(
                    y_hbm.at[srcbuf.at[js * K + i]],
                    buf.at[pl.ds(i * CHUNK, CHUNK)], sem).wait()

        def fire_scatters(js, buf, sem):
            for i in range(K):
                pltpu.async_copy(
                    buf.at[pl.ds(i * CHUNK, CHUNK)],
                    acc_s.at[dstbuf.at[js * K + i]], sem, add=True)

        def drain_scatters(js, buf, sem):
            for i in range(K):
                pltpu.make_async_copy(
                    buf.at[pl.ds(i * CHUNK, CHUNK)],
                    acc_s.at[dstbuf.at[js * K + i]], sem).wait()

        ring = ((rows_a, sem_ga, sem_sa),
                (rows_b, sem_gb, sem_sb),
                (rows_c, sem_gc, sem_sc))

        def step(j, cur, prev, drain_prev=True, fire_next=True):
            drain_gathers(j, cur[0], cur[1])
            fire_scatters(j, cur[0], cur[2])
            if drain_prev:
                drain_scatters(j - 1, prev[0], prev[2])
            if fire_next:
                fire_gathers(j + 2, prev[0], prev[1])

        fire_gathers(0, rows_a, sem_ga)
        fire_gathers(1, rows_b, sem_gb)
        step(0, ring[0], ring[2], drain_prev=False)
        step(1, ring[1], ring[0])
        step(2, ring[2], ring[1])

        def triple(g, _):
            j0 = 3 * g
            step(j0, ring[0], ring[2])
            step(j0 + 1, ring[1], ring[0])
            step(j0 + 2, ring[2], ring[1])
            return 0

        lax.fori_loop(1, nsuper // 3 - 1, triple, 0)
        j0 = nsuper - 3
        step(j0, ring[0], ring[2])
        step(j0 + 1, ring[1], ring[0], fire_next=False)
        step(j0 + 2, ring[2], ring[1], fire_next=False)
        drain_scatters(nsuper - 1, ring[2][0], ring[2][2])

        @pl.when(wid < extra)
        def _():
            pltpu.async_copy(
                y_hbm.at[srcbuf.at[base]],
                rows_a.at[pl.ds(0, CHUNK)], sem_ga).wait()
            pltpu.sync_copy(rows_a.at[pl.ds(0, CHUNK)],
                            acc_s.at[dstbuf.at[base]], add=True)

        plsc.subcore_barrier()

        pltpu.sync_copy(
            acc_s.at[pl.ds(s * rpt, rpt)],
            acc_out.at[c, pl.ds(s * rpt, rpt)],
        )

    return msg_kernel(y, edge3)


def _tc_matmul(x, w):
    n = x.shape[0]
    out_ch = w.shape[1]

    def body(x_ref, w_ref, xw_ref):
        xw_ref[...] = jnp.dot(x_ref[...], w_ref[...],
                              preferred_element_type=jnp.float32)

    return pl.pallas_call(
        body,
        out_shape=jax.ShapeDtypeStruct((n, out_ch), jnp.float32),
    )(x, w)


def _tc_scale(xw, deg_part, n):
    out_ch = xw.shape[1]

    def body(xw_ref, deg_ref, y_ref, dinv_ref):
        deg = deg_ref[0, :n, 0:1] + deg_ref[1, :n, 0:1]
        dinv = 1.0 / jnp.sqrt(deg + 1.0)
        y_ref[...] = xw_ref[...] * dinv
        dinv_ref[...] = dinv

    return pl.pallas_call(
        body,
        out_shape=[
            jax.ShapeDtypeStruct((n, out_ch), jnp.float32),
            jax.ShapeDtypeStruct((n, 1), jnp.float32),
        ],
    )(xw, deg_part)


def _tc_final(acc_part, y, dinv, b2, n):
    out_ch = acc_part.shape[2]

    def body(acc_ref, y_ref, dinv_ref, b_ref, o_ref):
        p = acc_ref[0, :n, :] + acc_ref[1, :n, :] + y_ref[...]
        o_ref[...] = p * dinv_ref[...] + b_ref[...]

    return pl.pallas_call(
        body,
        out_shape=jax.ShapeDtypeStruct((n, out_ch), jnp.float32),
    )(acc_part, y, dinv, b2)


def kernel(x, edge_index, W, b):
    n = x.shape[0]
    out_ch = W.shape[1]
    e = edge_index.shape[1]
    assert e % CHUNK == 0

    acc_rows = _round_up(n, NS * CHUNK)
    nch = e // CHUNK
    edge3 = edge_index.reshape(2, nch, CHUNK)

    deg_part = _sc_degree(edge3, acc_rows)
    xw = _tc_matmul(x, W)
    y, dinv = _tc_scale(xw, deg_part, n)
    acc_part = _sc_messages(y, edge3, acc_rows, out_ch)
    out = _tc_final(acc_part, y, dinv, b.reshape(1, out_ch), n)
    return out
